# Initial kernel scaffold; baseline (speedup 1.0000x reference)
import jax
import jax.numpy as jnp
from jax.experimental import pallas as pl


def kernel(x, W, b, mask_token):
    raise NotImplementedError("write your pallas kernel here")


# speedup vs baseline: 2.6937x; 2.6937x over previous
import functools
import math

import jax
import jax.numpy as jnp
from jax.experimental import pallas as pl
from jax.experimental.pallas import tpu as pltpu

_PATCH = 16
_EMBED = 768
_HW = 512
_GRID = _HW // _PATCH
_N = _GRID * _GRID
_WINDOW = 7
_NWIN = 4
_MASK_RATIO = 0.8
_RT = 256
_NG = _N // _RT


def _mask_array():
    H = W_ = _GRID
    all_inds = jnp.arange(H * W_, dtype=jnp.int32).reshape(H, W_)
    pad = _WINDOW // 2
    selectable = all_inds[pad:-pad, pad:-pad].reshape(-1)
    key = jax.random.key(42)
    sampled = jax.random.choice(key, selectable.shape[0], (_NWIN,), replace=False)
    centroids = selectable[sampled]
    off = jnp.arange(int(math.ceil(-_WINDOW / 2)), int(math.ceil(_WINDOW / 2)),
                     dtype=jnp.int32)
    wo = jnp.tile(off[None, :], (_WINDOW, 1))
    sq = jnp.tile((off * H)[None, :], (_WINDOW, 1)).T
    wo = (wo + sq).reshape(1, -1)
    coords = jnp.tile(centroids[:, None], (1, _WINDOW ** 2)) + wo
    n_mask = int(_MASK_RATIO * _WINDOW ** 2)
    inds = coords[:, :n_mask]
    mask = jnp.zeros((_NWIN, _N), jnp.float32)
    mask = mask.at[jnp.arange(_NWIN)[:, None], inds].set(1.0)
    return mask.T.reshape(_NG, _RT, _NWIN)


def _body(xp_ref, w_ref, b_ref, tok_ref, mask_ref, out_ref):
    acc = jnp.dot(xp_ref[0], w_ref[...], preferred_element_type=jnp.float32)
    acc = acc + b_ref[...]
    tok = jnp.broadcast_to(tok_ref[...], acc.shape)
    m = mask_ref[0]
    for w in range(_NWIN):
        sel = jnp.broadcast_to(m[:, w:w + 1] != 0.0, acc.shape)
        out_ref[0, w] = jnp.where(sel, tok, acc)


def kernel(x, W, b, mask_token):
    Bn = x.shape[0]
    p = _PATCH
    xp = x.reshape(Bn, 3, _GRID, p, _GRID, p)
    xp = xp.transpose(0, 2, 4, 1, 3, 5).reshape(Bn, _N, 3 * p * p)
    Wm = W.reshape(_EMBED, 3 * p * p).T
    mask = _mask_array()
    tok = mask_token.reshape(1, _EMBED)
    b2 = b.reshape(1, _EMBED)

    out = pl.pallas_call(
        _body,
        grid=(Bn, _NG),
        in_specs=[
            pl.BlockSpec((1, _RT, 3 * p * p), lambda bi, i: (bi, i, 0)),
            pl.BlockSpec((3 * p * p, _EMBED), lambda bi, i: (0, 0)),
            pl.BlockSpec((1, _EMBED), lambda bi, i: (0, 0)),
            pl.BlockSpec((1, _EMBED), lambda bi, i: (0, 0)),
            pl.BlockSpec((1, _RT, _NWIN), lambda bi, i: (i, 0, 0)),
        ],
        out_specs=pl.BlockSpec((1, _NWIN, _RT, _EMBED),
                               lambda bi, i: (bi, 0, i, 0)),
        out_shape=jax.ShapeDtypeStruct((Bn, _NWIN, _N, _EMBED), jnp.float32),
        compiler_params=pltpu.CompilerParams(
            dimension_semantics=("parallel", "parallel")),
    )(xp, Wm, b2, tok, mask)
    return out
